# baseline (speedup 1.0000x reference)
import functools

import jax
import jax.numpy as jnp
import numpy as np
from jax import lax
from jax.experimental import pallas as pl
from jax.experimental.pallas import tpu as pltpu
from jax.experimental.pallas import tpu_sc as plsc

N_NODES = 10000
N_EDGES = 320000
HIDDEN = 128
GRAPHS = 32
LAYERS = 2

NC = 2
NS = 16
NW = NC * NS

CH = 128

EMB_CHUNKS = 3
EMB_PER_W = EMB_CHUNKS * CH
N_PAD = NW * EMB_PER_W

EDGE_CHUNKS = 80
EDGE_PER_W = EDGE_CHUNKS * CH
E_PAD = NW * EDGE_PER_W
NBUF = 2

ACC_ROWS = 10112
ZROWS = ACC_ROWS // NS
CPROWS = 624
CPREM = N_NODES - NS * CPROWS

_sc_mesh = plsc.VectorSubcoreMesh(core_axis_name="c", subcore_axis_name="s")


NI = 4


EC2 = N_PAD // (NS * CH)
EPW2 = EC2 * CH


@functools.partial(
    pl.kernel,
    out_type=[jax.ShapeDtypeStruct((NC, N_NODES, HIDDEN), jnp.float32),
              jax.ShapeDtypeStruct((N_PAD, HIDDEN), jnp.float32)],
    mesh=_sc_mesh,
    scratch_types=(
        [pltpu.VMEM_SHARED((ACC_ROWS, HIDDEN), jnp.float32)]
        + [pltpu.VMEM((2, CH), jnp.int32)] * NI
        + [pltpu.VMEM((CH,), jnp.int32)] * EC2
        + [pltpu.VMEM((CH, HIDDEN), jnp.float32)] * NBUF
        + [pltpu.SemaphoreType.DMA] * NI
        + [pltpu.SemaphoreType.DMA] * NBUF
        + [pltpu.SemaphoreType.DMA, pltpu.SemaphoreType.DMA]
    ),
)
def _edge_scatter_emb(emb_hbm, x_hbm, ei_hbm, zeros_hbm, out_hbm, h_hbm,
                      accum, *bufs):
    idxb = bufs[:NI]
    xsb = bufs[NI:NI + EC2]
    rows = bufs[NI + EC2:NI + EC2 + NBUF]
    isem = bufs[NI + EC2 + NBUF:2 * NI + EC2 + NBUF]
    gsem = bufs[2 * NI + EC2 + NBUF:2 * NI + EC2 + 2 * NBUF]
    esem = bufs[2 * NI + EC2 + 2 * NBUF]
    zsem = bufs[2 * NI + EC2 + 2 * NBUF + 1]
    c = lax.axis_index("c")
    s = lax.axis_index("s")
    wid = s * NC + c

    zslice = accum.at[pl.ds(pl.multiple_of(s * ZROWS, 8), ZROWS)]
    zdesc = pltpu.async_copy(zeros_hbm, zslice, zsem)

    for t in range(NI):
        pltpu.async_copy(ei_hbm.at[wid, t], idxb[t], isem[t])

    hb = pl.multiple_of(s * EPW2, CH)
    for k in range(EC2):
        pltpu.async_copy(x_hbm.at[pl.ds(hb + k * CH, CH)], xsb[k], esem)
    for k in range(EC2):
        pltpu.make_async_copy(x_hbm.at[pl.ds(hb + k * CH, CH)], xsb[k],
                              esem).wait()
    for b in range(NBUF):
        pltpu.async_copy(emb_hbm.at[xsb[b]], rows[b], gsem[b])
    for k in range(EC2):
        b = k % NBUF
        pltpu.make_async_copy(emb_hbm.at[xsb[k]], rows[b], gsem[b]).wait()
        pltpu.sync_copy(rows[b], h_hbm.at[pl.ds(hb + k * CH, CH)])
        if k + NBUF < EC2:
            pltpu.async_copy(emb_hbm.at[xsb[k + NBUF]], rows[b], gsem[b])

    zdesc.wait()
    plsc.subcore_barrier()

    for b in range(NBUF):
        pltpu.make_async_copy(ei_hbm.at[wid, b], idxb[b], isem[b]).wait()
        pltpu.async_copy(h_hbm.at[idxb[b].at[0]], rows[b], gsem[b])

    def step(o, carry):
        for t in range(NI):
            j = o * NI + t
            b = t % NBUF
            pltpu.make_async_copy(h_hbm.at[idxb[t].at[0]], rows[b],
                                  gsem[b]).wait()
            pltpu.sync_copy(rows[b], accum.at[idxb[t].at[1]], add=True)

            @pl.when(j + NI < EDGE_CHUNKS)
            def _reidx():
                pltpu.async_copy(ei_hbm.at[wid, j + NI], idxb[t], isem[t])

            @pl.when(j + NBUF < EDGE_CHUNKS)
            def _regather():
                tg = (t + NBUF) % NI
                pltpu.make_async_copy(ei_hbm.at[wid, j + NBUF], idxb[tg],
                                      isem[tg]).wait()
                pltpu.async_copy(h_hbm.at[idxb[tg].at[0]], rows[b],
                                 gsem[b])
        return carry

    lax.fori_loop(0, EDGE_CHUNKS // NI, step, 0)
    plsc.subcore_barrier()

    r0 = pl.multiple_of(s * CPROWS, 8)
    pltpu.sync_copy(accum.at[pl.ds(r0, CPROWS)],
                    out_hbm.at[c, pl.ds(r0, CPROWS)])

    @pl.when(s == 0)
    def _rem0():
        pltpu.sync_copy(accum.at[pl.ds(NS * CPROWS, CPREM)],
                        out_hbm.at[c, pl.ds(NS * CPROWS, CPREM)])


@functools.partial(
    pl.kernel,
    out_type=jax.ShapeDtypeStruct((NC, N_NODES, HIDDEN), jnp.float32),
    mesh=_sc_mesh,
    scratch_types=(
        [pltpu.VMEM_SHARED((ACC_ROWS, HIDDEN), jnp.float32)]
        + [pltpu.VMEM((2, CH), jnp.int32)] * NI
        + [pltpu.VMEM((CH, HIDDEN), jnp.float32)] * NBUF
        + [pltpu.SemaphoreType.DMA] * NI
        + [pltpu.SemaphoreType.DMA] * NBUF
        + [pltpu.SemaphoreType.DMA, pltpu.SemaphoreType.DMA]
    ),
)
def _edge_scatter(h_hbm, ei_hbm, zeros_hbm, out_hbm, accum, *bufs):
    idxb = bufs[:NI]
    rows = bufs[NI:NI + NBUF]
    isem = bufs[NI + NBUF:2 * NI + NBUF]
    gsem = bufs[2 * NI + NBUF:2 * NI + 2 * NBUF]
    ssem = bufs[2 * NI + 2 * NBUF]
    zsem = bufs[2 * NI + 2 * NBUF + 1]
    c = lax.axis_index("c")
    s = lax.axis_index("s")
    wid = s * NC + c

    zslice = accum.at[pl.ds(pl.multiple_of(s * ZROWS, 8), ZROWS)]
    zdesc = pltpu.async_copy(zeros_hbm, zslice, zsem)

    for t in range(NI):
        pltpu.async_copy(ei_hbm.at[wid, t], idxb[t], isem[t])
    for b in range(NBUF):
        pltpu.make_async_copy(ei_hbm.at[wid, b], idxb[b], isem[b]).wait()
        pltpu.async_copy(h_hbm.at[idxb[b].at[0]], rows[b], gsem[b])

    zdesc.wait()
    plsc.subcore_barrier()

    def step(o, carry):
        for t in range(NI):
            j = o * NI + t
            b = t % NBUF
            pltpu.make_async_copy(h_hbm.at[idxb[t].at[0]], rows[b],
                                  gsem[b]).wait()
            pltpu.sync_copy(rows[b], accum.at[idxb[t].at[1]], add=True)

            @pl.when(j + NI < EDGE_CHUNKS)
            def _reidx():
                pltpu.async_copy(ei_hbm.at[wid, j + NI], idxb[t], isem[t])

            @pl.when(j + NBUF < EDGE_CHUNKS)
            def _regather():
                tn = (t + NBUF) % NI
                pltpu.make_async_copy(ei_hbm.at[wid, j + NBUF], idxb[tn],
                                      isem[tn]).wait()
                pltpu.async_copy(h_hbm.at[idxb[tn].at[0]], rows[b],
                                 gsem[b])
        return carry

    lax.fori_loop(0, EDGE_CHUNKS // NI, step, 0)
    plsc.subcore_barrier()

    r0 = pl.multiple_of(s * CPROWS, 8)
    pltpu.sync_copy(accum.at[pl.ds(r0, CPROWS)],
                    out_hbm.at[c, pl.ds(r0, CPROWS)])

    @pl.when(s == 0)
    def _rem():
        pltpu.sync_copy(accum.at[pl.ds(NS * CPROWS, CPREM)],
                        out_hbm.at[c, pl.ds(NS * CPROWS, CPREM)])


_GRID_R = 1000


def _gru_block(p0, p1, h, wih, whh, bih, bhh):
    xn = p0 + p1
    gi = jnp.dot(xn, wih, preferred_element_type=jnp.float32) + bih
    gh = jnp.dot(h, whh, preferred_element_type=jnp.float32) + bhh
    r = jax.nn.sigmoid(gi[:, :HIDDEN] + gh[:, :HIDDEN])
    z = jax.nn.sigmoid(gi[:, HIDDEN:2 * HIDDEN] + gh[:, HIDDEN:2 * HIDDEN])
    n = jnp.tanh(gi[:, 2 * HIDDEN:] + r * gh[:, 2 * HIDDEN:])
    return (1.0 - z) * n + z * h


def _gru_body(p0_ref, p1_ref, h_ref, wih_ref, whh_ref, bih_ref, bhh_ref,
              out_ref):
    out_ref[...] = _gru_block(p0_ref[0], p1_ref[0], h_ref[...],
                              wih_ref[...], whh_ref[...],
                              bih_ref[...], bhh_ref[...])


_P_SPEC0 = pl.BlockSpec((1, 1000, HIDDEN), lambda i: (0, i, 0))
_P_SPEC1 = pl.BlockSpec((1, 1000, HIDDEN), lambda i: (1, i, 0))


def _gru_tc(part, h, wih_t, whh_t, bih, bhh):
    grid = (N_NODES // _GRID_R,)
    blk = lambda i: (i, 0)
    whole = lambda i: (0, 0)
    return pl.pallas_call(
        _gru_body,
        grid=grid,
        in_specs=[
            _P_SPEC0,
            _P_SPEC1,
            pl.BlockSpec((_GRID_R, HIDDEN), blk),
            pl.BlockSpec((HIDDEN, 3 * HIDDEN), whole),
            pl.BlockSpec((HIDDEN, 3 * HIDDEN), whole),
            pl.BlockSpec((1, 3 * HIDDEN), whole),
            pl.BlockSpec((1, 3 * HIDDEN), whole),
        ],
        out_specs=pl.BlockSpec((_GRID_R, HIDDEN), blk),
        out_shape=jax.ShapeDtypeStruct((N_NODES, HIDDEN), jnp.float32),
    )(part, part, h, wih_t, whh_t, bih, bhh)


def _gru_tail_body(p0_ref, p1_ref, h_ref, wih_ref, whh_ref, bih_ref,
                   bhh_ref, bat_ref, dw_ref, db_ref, cw_ref, cb_ref,
                   out_ref, pooled_ref):
    i = pl.program_id(0)

    @pl.when(i == 0)
    def _init():
        pooled_ref[...] = jnp.full((GRAPHS, HIDDEN), -jnp.inf,
                                   dtype=jnp.float32)

    hn = _gru_block(p0_ref[0], p1_ref[0], h_ref[...],
                    wih_ref[...], whh_ref[...],
                    bih_ref[...], bhh_ref[...])
    hd = jnp.dot(hn, dw_ref[...], preferred_element_type=jnp.float32)
    hd = hd + db_ref[...]
    bat = bat_ref[...]
    neg = jnp.float32(-jnp.inf)
    zero = jnp.float32(0.0)
    for g in range(GRAPHS):
        madd = jnp.where(bat == g, zero, neg)
        m = (hd + madd).max(axis=0, keepdims=True)
        pooled_ref[g:g + 1, :] = jnp.maximum(pooled_ref[g:g + 1, :], m)

    @pl.when(i == pl.num_programs(0) - 1)
    def _fin():
        logits = jnp.dot(pooled_ref[...], cw_ref[...],
                         preferred_element_type=jnp.float32) + cb_ref[...]
        out_ref[...] = jax.nn.sigmoid(logits)


def _gru_tail_tc(part, h, wih_t, whh_t, bih, bhh, bat2d, dw_t, db, cw_t, cb):
    grid = (N_NODES // _GRID_R,)
    blk = lambda i: (i, 0)
    whole = lambda i: (0, 0)
    return pl.pallas_call(
        _gru_tail_body,
        grid=grid,
        in_specs=[
            _P_SPEC0,
            _P_SPEC1,
            pl.BlockSpec((_GRID_R, HIDDEN), blk),
            pl.BlockSpec((HIDDEN, 3 * HIDDEN), whole),
            pl.BlockSpec((HIDDEN, 3 * HIDDEN), whole),
            pl.BlockSpec((1, 3 * HIDDEN), whole),
            pl.BlockSpec((1, 3 * HIDDEN), whole),
            pl.BlockSpec((_GRID_R, 1), blk),
            pl.BlockSpec((HIDDEN, HIDDEN), whole),
            pl.BlockSpec((1, HIDDEN), whole),
            pl.BlockSpec((HIDDEN, 1), whole),
            pl.BlockSpec((1, 1), whole),
        ],
        out_specs=pl.BlockSpec((GRAPHS, 1), whole),
        out_shape=jax.ShapeDtypeStruct((GRAPHS, 1), jnp.float32),
        scratch_shapes=[pltpu.VMEM((GRAPHS, HIDDEN), jnp.float32)],
    )(part, part, h, wih_t, whh_t, bih, bhh, bat2d, dw_t, db, cw_t, cb)


def kernel(x, edge_index, batch, emb, W_ih, W_hh, b_ih, b_hh,
           dense_W, dense_b, clf_W, clf_b):
    x_pad = jnp.concatenate(
        [x, jnp.zeros((N_PAD - N_NODES,), jnp.int32)])

    npad = E_PAD - N_EDGES
    pad_iota = np.arange(npad, dtype=np.int32)
    pad_src = jnp.asarray(pad_iota % N_NODES)
    pad_dst = jnp.asarray(N_NODES + pad_iota % (ACC_ROWS - N_NODES))
    src = jnp.concatenate(
        [edge_index[0], pad_src]).reshape(NW, EDGE_CHUNKS, 1, CH)
    dst = jnp.concatenate(
        [edge_index[1], pad_dst]).reshape(NW, EDGE_CHUNKS, 1, CH)
    ei = jnp.concatenate([src, dst], axis=2)
    zeros = jnp.zeros((ZROWS, HIDDEN), jnp.float32)

    part, h_pad = _edge_scatter_emb(emb, x_pad, ei, zeros)
    h = _gru_tc(part, h_pad, W_ih[0].T, W_hh[0].T,
                b_ih[0][None, :], b_hh[0][None, :])
    part = _edge_scatter(h, ei, zeros)
    out2 = _gru_tail_tc(part, h, W_ih[1].T, W_hh[1].T,
                        b_ih[1][None, :], b_hh[1][None, :],
                        batch[:, None], dense_W.T, dense_b[None, :],
                        clf_W.T, clf_b[None, :])
    return out2[:, 0]


# speedup vs baseline: 1.2752x; 1.2752x over previous
import functools

import jax
import jax.numpy as jnp
import numpy as np
from jax import lax
from jax.experimental import pallas as pl
from jax.experimental.pallas import tpu as pltpu
from jax.experimental.pallas import tpu_sc as plsc

N_NODES = 10000
N_EDGES = 320000
HIDDEN = 128
GRAPHS = 32
LAYERS = 2

NC = 2
NS = 16
NW = NC * NS

CH = 128

EMB_CHUNKS = 3
EMB_PER_W = EMB_CHUNKS * CH
N_PAD = NW * EMB_PER_W

EDGE_CHUNKS = 80
EDGE_PER_W = EDGE_CHUNKS * CH
E_PAD = NW * EDGE_PER_W
NBUF = 2

ACC_ROWS = 10112
ZROWS = ACC_ROWS // NS
CPROWS = 624
CPREM = N_NODES - NS * CPROWS

_sc_mesh = plsc.VectorSubcoreMesh(core_axis_name="c", subcore_axis_name="s")


NI = 4


@functools.partial(
    pl.kernel,
    out_type=jax.ShapeDtypeStruct((N_PAD, HIDDEN), jnp.float32),
    mesh=_sc_mesh,
    scratch_types=(
        [pltpu.VMEM((EMB_CHUNKS, CH), jnp.int32)]
        + [pltpu.VMEM((CH, HIDDEN), jnp.float32)] * EMB_CHUNKS
        + [pltpu.SemaphoreType.DMA] * EMB_CHUNKS
        + [pltpu.SemaphoreType.DMA]
    ),
)
def _emb_gather(emb_hbm, idx_hbm, out_hbm, idx_v, *bufs):
    rows = bufs[:EMB_CHUNKS]
    gsem = bufs[EMB_CHUNKS:2 * EMB_CHUNKS]
    wsem = bufs[2 * EMB_CHUNKS]
    c = lax.axis_index("c")
    s = lax.axis_index("s")
    wid = s * NC + c
    base = wid * EMB_PER_W

    pltpu.sync_copy(idx_hbm.at[wid], idx_v)
    for b in range(EMB_CHUNKS):
        pltpu.async_copy(emb_hbm.at[idx_v.at[b]], rows[b], gsem[b])
    for b in range(EMB_CHUNKS):
        o = pl.multiple_of(base + b * CH, CH)
        pltpu.make_async_copy(emb_hbm.at[idx_v.at[b]], rows[b],
                              gsem[b]).wait()
        pltpu.async_copy(rows[b], out_hbm.at[pl.ds(o, CH)], wsem)
    for b in range(EMB_CHUNKS):
        o = pl.multiple_of(base + b * CH, CH)
        pltpu.make_async_copy(rows[b], out_hbm.at[pl.ds(o, CH)],
                              wsem).wait()


@functools.partial(
    pl.kernel,
    out_type=jax.ShapeDtypeStruct((NC, N_NODES, HIDDEN), jnp.float32),
    mesh=_sc_mesh,
    scratch_types=(
        [pltpu.VMEM_SHARED((ACC_ROWS, HIDDEN), jnp.float32)]
        + [pltpu.VMEM((2, CH), jnp.int32)] * NI
        + [pltpu.VMEM((CH, HIDDEN), jnp.float32)] * NBUF
        + [pltpu.SemaphoreType.DMA] * NI
        + [pltpu.SemaphoreType.DMA] * NBUF
        + [pltpu.SemaphoreType.DMA, pltpu.SemaphoreType.DMA]
    ),
)
def _edge_scatter(h_hbm, ei_hbm, zeros_hbm, out_hbm, accum, *bufs):
    idxb = bufs[:NI]
    rows = bufs[NI:NI + NBUF]
    isem = bufs[NI + NBUF:2 * NI + NBUF]
    gsem = bufs[2 * NI + NBUF:2 * NI + 2 * NBUF]
    ssem = bufs[2 * NI + 2 * NBUF]
    zsem = bufs[2 * NI + 2 * NBUF + 1]
    c = lax.axis_index("c")
    s = lax.axis_index("s")
    wid = s * NC + c

    zslice = accum.at[pl.ds(pl.multiple_of(s * ZROWS, 8), ZROWS)]
    zdesc = pltpu.async_copy(zeros_hbm, zslice, zsem)

    for t in range(NI):
        pltpu.async_copy(ei_hbm.at[wid, t], idxb[t], isem[t])
    for b in range(NBUF):
        pltpu.make_async_copy(ei_hbm.at[wid, b], idxb[b], isem[b]).wait()
        pltpu.async_copy(h_hbm.at[idxb[b].at[0]], rows[b], gsem[b])

    zdesc.wait()
    plsc.subcore_barrier()

    def step(o, carry):
        for t in range(NI):
            j = o * NI + t
            b = t % NBUF
            pltpu.make_async_copy(h_hbm.at[idxb[t].at[0]], rows[b],
                                  gsem[b]).wait()
            pltpu.sync_copy(rows[b], accum.at[idxb[t].at[1]], add=True)

            @pl.when(j + NI < EDGE_CHUNKS)
            def _reidx():
                pltpu.async_copy(ei_hbm.at[wid, j + NI], idxb[t], isem[t])

            @pl.when(j + NBUF < EDGE_CHUNKS)
            def _regather():
                tn = (t + NBUF) % NI
                pltpu.make_async_copy(ei_hbm.at[wid, j + NBUF], idxb[tn],
                                      isem[tn]).wait()
                pltpu.async_copy(h_hbm.at[idxb[tn].at[0]], rows[b],
                                 gsem[b])
        return carry

    lax.fori_loop(0, EDGE_CHUNKS // NI, step, 0)
    plsc.subcore_barrier()

    r0 = pl.multiple_of(s * CPROWS, 8)
    pltpu.sync_copy(accum.at[pl.ds(r0, CPROWS)],
                    out_hbm.at[c, pl.ds(r0, CPROWS)])

    @pl.when(s == 0)
    def _rem():
        pltpu.sync_copy(accum.at[pl.ds(NS * CPROWS, CPREM)],
                        out_hbm.at[c, pl.ds(NS * CPROWS, CPREM)])


_GRID_R = 1000


def _gru_block(p0, p1, h, wih, whh, bih, bhh):
    xn = p0 + p1
    gi = jnp.dot(xn, wih, preferred_element_type=jnp.float32) + bih
    gh = jnp.dot(h, whh, preferred_element_type=jnp.float32) + bhh
    r = jax.nn.sigmoid(gi[:, :HIDDEN] + gh[:, :HIDDEN])
    z = jax.nn.sigmoid(gi[:, HIDDEN:2 * HIDDEN] + gh[:, HIDDEN:2 * HIDDEN])
    n = jnp.tanh(gi[:, 2 * HIDDEN:] + r * gh[:, 2 * HIDDEN:])
    return (1.0 - z) * n + z * h


def _gru_body(p0_ref, p1_ref, h_ref, wih_ref, whh_ref, bih_ref, bhh_ref,
              out_ref):
    out_ref[...] = _gru_block(p0_ref[0], p1_ref[0], h_ref[...],
                              wih_ref[...], whh_ref[...],
                              bih_ref[...], bhh_ref[...])


_P_SPEC0 = pl.BlockSpec((1, 1000, HIDDEN), lambda i: (0, i, 0))
_P_SPEC1 = pl.BlockSpec((1, 1000, HIDDEN), lambda i: (1, i, 0))


def _gru_tc(part, h, wih_t, whh_t, bih, bhh):
    grid = (N_NODES // _GRID_R,)
    blk = lambda i: (i, 0)
    whole = lambda i: (0, 0)
    return pl.pallas_call(
        _gru_body,
        grid=grid,
        in_specs=[
            _P_SPEC0,
            _P_SPEC1,
            pl.BlockSpec((_GRID_R, HIDDEN), blk),
            pl.BlockSpec((HIDDEN, 3 * HIDDEN), whole),
            pl.BlockSpec((HIDDEN, 3 * HIDDEN), whole),
            pl.BlockSpec((1, 3 * HIDDEN), whole),
            pl.BlockSpec((1, 3 * HIDDEN), whole),
        ],
        out_specs=pl.BlockSpec((_GRID_R, HIDDEN), blk),
        out_shape=jax.ShapeDtypeStruct((N_NODES, HIDDEN), jnp.float32),
    )(part, part, h, wih_t, whh_t, bih, bhh)


def _gru_tail_body(p0_ref, p1_ref, h_ref, wih_ref, whh_ref, bih_ref,
                   bhh_ref, bat_ref, dw_ref, db_ref, cw_ref, cb_ref,
                   out_ref, pooled_ref):
    i = pl.program_id(0)

    @pl.when(i == 0)
    def _init():
        pooled_ref[...] = jnp.full((GRAPHS, HIDDEN), -jnp.inf,
                                   dtype=jnp.float32)

    hn = _gru_block(p0_ref[0], p1_ref[0], h_ref[...],
                    wih_ref[...], whh_ref[...],
                    bih_ref[...], bhh_ref[...])
    hd = jnp.dot(hn, dw_ref[...], preferred_element_type=jnp.float32)
    hd = hd + db_ref[...]
    bat = bat_ref[...]
    neg = jnp.float32(-jnp.inf)
    zero = jnp.float32(0.0)
    for g in range(GRAPHS):
        madd = jnp.where(bat == g, zero, neg)
        m = (hd + madd).max(axis=0, keepdims=True)
        pooled_ref[g:g + 1, :] = jnp.maximum(pooled_ref[g:g + 1, :], m)

    @pl.when(i == pl.num_programs(0) - 1)
    def _fin():
        logits = jnp.dot(pooled_ref[...], cw_ref[...],
                         preferred_element_type=jnp.float32) + cb_ref[...]
        out_ref[...] = jax.nn.sigmoid(logits)


def _gru_tail_tc(part, h, wih_t, whh_t, bih, bhh, bat2d, dw_t, db, cw_t, cb):
    grid = (N_NODES // _GRID_R,)
    blk = lambda i: (i, 0)
    whole = lambda i: (0, 0)
    return pl.pallas_call(
        _gru_tail_body,
        grid=grid,
        in_specs=[
            _P_SPEC0,
            _P_SPEC1,
            pl.BlockSpec((_GRID_R, HIDDEN), blk),
            pl.BlockSpec((HIDDEN, 3 * HIDDEN), whole),
            pl.BlockSpec((HIDDEN, 3 * HIDDEN), whole),
            pl.BlockSpec((1, 3 * HIDDEN), whole),
            pl.BlockSpec((1, 3 * HIDDEN), whole),
            pl.BlockSpec((_GRID_R, 1), blk),
            pl.BlockSpec((HIDDEN, HIDDEN), whole),
            pl.BlockSpec((1, HIDDEN), whole),
            pl.BlockSpec((HIDDEN, 1), whole),
            pl.BlockSpec((1, 1), whole),
        ],
        out_specs=pl.BlockSpec((GRAPHS, 1), whole),
        out_shape=jax.ShapeDtypeStruct((GRAPHS, 1), jnp.float32),
        scratch_shapes=[pltpu.VMEM((GRAPHS, HIDDEN), jnp.float32)],
    )(part, part, h, wih_t, whh_t, bih, bhh, bat2d, dw_t, db, cw_t, cb)


def kernel(x, edge_index, batch, emb, W_ih, W_hh, b_ih, b_hh,
           dense_W, dense_b, clf_W, clf_b):
    x_pad = jnp.concatenate(
        [x, jnp.zeros((N_PAD - N_NODES,), jnp.int32)])

    npad = E_PAD - N_EDGES
    pad_iota = np.arange(npad, dtype=np.int32)
    pad_src = jnp.asarray(pad_iota % N_NODES)
    pad_dst = jnp.asarray(N_NODES + pad_iota % (ACC_ROWS - N_NODES))
    src = jnp.concatenate(
        [edge_index[0], pad_src]).reshape(NW, EDGE_CHUNKS, 1, CH)
    dst = jnp.concatenate(
        [edge_index[1], pad_dst]).reshape(NW, EDGE_CHUNKS, 1, CH)
    ei = jnp.concatenate([src, dst], axis=2)
    zeros = jnp.zeros((ZROWS, HIDDEN), jnp.float32)

    h_pad = _emb_gather(emb, x_pad.reshape(NW, EMB_CHUNKS, CH))
    part = _edge_scatter(h_pad, ei, zeros)
    h = _gru_tc(part, h_pad, W_ih[0].T, W_hh[0].T,
                b_ih[0][None, :], b_hh[0][None, :])
    part = _edge_scatter(h, ei, zeros)
    out2 = _gru_tail_tc(part, h, W_ih[1].T, W_hh[1].T,
                        b_ih[1][None, :], b_hh[1][None, :],
                        batch[:, None], dense_W.T, dense_b[None, :],
                        clf_W.T, clf_b[None, :])
    return out2[:, 0]
